# baseline (speedup 1.0000x reference)
import functools

import jax
import jax.numpy as jnp
from jax import lax
from jax.experimental import pallas as pl
from jax.experimental.pallas import tpu as pltpu
from jax.experimental.pallas import tpu_sc as plsc

NC, NS, L = 2, 16, 16
NW = NC * NS
ROWS, COLS = 128, 32768
RPW = ROWS // NW
UNROLL = 4
NVREG = COLS // L
NITER = NVREG // UNROLL

def _permute(v, idx):
    dnums = lax.GatherDimensionNumbers(
        offset_dims=(), collapsed_slice_dims=(0,), start_index_map=(0,)
    )
    return lax.gather(
        v, idx[:, None], dnums, (1,),
        mode=lax.GatherScatterMode.PROMISE_IN_BOUNDS,
    )


_mesh = plsc.VectorSubcoreMesh(
    core_axis_name="c", subcore_axis_name="s", num_cores=NC, num_subcores=NS
)


@functools.partial(
    pl.kernel,
    out_type=jax.ShapeDtypeStruct((NW, L), jnp.int32),
    mesh=_mesh,
    scratch_types=[
        pltpu.VMEM((COLS,), jnp.float32),
        pltpu.VMEM((COLS,), jnp.float32),
        pltpu.VMEM((L,), jnp.int32),
        pltpu.SemaphoreType.DMA,
        pltpu.SemaphoreType.DMA,
    ],
)
def _argmin_rows_sc(x_hbm, out_hbm, buf0, buf1, res_v, sem0, sem1):
    wid = lax.axis_index("s") * NC + lax.axis_index("c")
    row0 = wid * RPW
    bufs = (buf0, buf1)
    sems = (sem0, sem1)
    lane = lax.iota(jnp.int32, 16)

    copies = [None] * RPW
    copies[0] = pltpu.async_copy(x_hbm.at[row0], buf0, sem0)

    res = jnp.zeros((L,), jnp.int32)
    for r in range(RPW):
        if r + 1 < RPW:
            copies[r + 1] = pltpu.async_copy(
                x_hbm.at[row0 + r + 1], bufs[(r + 1) % 2], sems[(r + 1) % 2]
            )
        copies[r].wait()
        buf = bufs[r % 2]

        def body(i, carry, buf=buf):
            ms, ids = list(carry[0]), list(carry[1])
            base = i * (UNROLL * L)
            for k in range(UNROLL):
                v = buf[pl.ds(base + k * L, L)]
                vi = jnp.full((L,), i * UNROLL + k, jnp.int32)
                pred = v < ms[k]
                ms[k] = jnp.where(pred, v, ms[k])
                ids[k] = jnp.where(pred, vi, ids[k])
            return tuple(ms), tuple(ids)

        inf = jnp.full((L,), jnp.inf, jnp.float32)
        zero = jnp.zeros((L,), jnp.int32)
        ms, ids = lax.fori_loop(
            0, NITER, body, ((inf,) * UNROLL, (zero,) * UNROLL)
        )

        m, g = ms[0], ids[0]
        for k in range(1, UNROLL):
            pred = (ms[k] < m) | ((ms[k] == m) & (ids[k] < g))
            m = jnp.where(pred, ms[k], m)
            g = jnp.where(pred, ids[k], g)

        e = g * L + lane
        for sh in (8, 4, 2, 1):
            perm = lane ^ sh
            mp = _permute(m, perm)
            ep = _permute(e, perm)
            pred = (mp < m) | ((mp == m) & (ep < e))
            m = jnp.where(pred, mp, m)
            e = jnp.where(pred, ep, e)
        res = jnp.where(lane == r, e, res)

    res_v[...] = res
    pltpu.sync_copy(res_v, out_hbm.at[wid])


def kernel(x):
    out = _argmin_rows_sc(x)
    return out[:, :RPW].reshape(ROWS)
